# Initial kernel scaffold; baseline (speedup 1.0000x reference)
import jax
import jax.numpy as jnp
from jax.experimental import pallas as pl


def kernel(x, v1, g1, b1, v2, g2, b2, w_down, b_down):
    raise NotImplementedError("write your pallas kernel here")


# speedup vs baseline: 2.0965x; 2.0965x over previous
import functools

import jax
import jax.numpy as jnp
from jax.experimental import pallas as pl
from jax.experimental.pallas import tpu as pltpu


def _wn(v, g):
    norm = jnp.sqrt(jnp.sum(v.astype(jnp.float32) ** 2, axis=(1, 2), keepdims=True))
    return (g.astype(jnp.float32) * v.astype(jnp.float32) / norm)


def _shift_right(a, s):
    if s == 0:
        return a
    rows = a.shape[0]
    return jnp.concatenate(
        [jnp.zeros((rows, s), a.dtype), a[:, :a.shape[1] - s]], axis=1)


def _causal_tap_matmul(w_ref, a, kernel_size, dilation):
    pad = dilation * (kernel_size - 1)
    acc = jnp.dot(w_ref[kernel_size - 1], a, preferred_element_type=jnp.float32)
    for k in range(kernel_size - 1):
        acc = acc + jnp.dot(w_ref[k], _shift_right(a, pad - k * dilation),
                            preferred_element_type=jnp.float32)
    return acc


def _fused_block_kernel(x_ref, w1_ref, b1_ref, w2_ref, wd_ref, b2_ref, bd_ref,
                        o_ref, *, kernel_size, dilation):
    xb = x_ref[0].astype(jnp.bfloat16)

    acc1 = _causal_tap_matmul(w1_ref, xb, kernel_size, dilation)
    h1 = jnp.maximum(acc1 + b1_ref[...], 0.0).astype(jnp.bfloat16)

    acc2 = _causal_tap_matmul(w2_ref, h1, kernel_size, dilation)
    h2 = jnp.maximum(acc2 + b2_ref[...], 0.0)

    res = jnp.dot(wd_ref[...], xb, preferred_element_type=jnp.float32)
    o_ref[0] = jnp.maximum(h2 + res + bd_ref[...], 0.0).astype(o_ref.dtype)


def kernel(x, v1, g1, b1, v2, g2, b2, w_down, b_down):
    n, c_in, l = x.shape
    c_out = v1.shape[0]
    k = v1.shape[2]
    dilation = 2

    w1 = jnp.transpose(_wn(v1, g1), (2, 0, 1)).astype(jnp.bfloat16)
    w2 = jnp.transpose(_wn(v2, g2), (2, 0, 1)).astype(jnp.bfloat16)
    wd = w_down.reshape(c_out, c_in).astype(jnp.bfloat16)
    b1c = b1.astype(jnp.float32).reshape(c_out, 1)
    b2c = b2.astype(jnp.float32).reshape(c_out, 1)
    bdc = b_down.astype(jnp.float32).reshape(c_out, 1)

    kern = functools.partial(_fused_block_kernel, kernel_size=k,
                             dilation=dilation)
    return pl.pallas_call(
        kern,
        out_shape=jax.ShapeDtypeStruct((n, c_out, l), x.dtype),
        grid=(n,),
        in_specs=[
            pl.BlockSpec((1, c_in, l), lambda i: (i, 0, 0)),
            pl.BlockSpec((k, c_out, c_in), lambda i: (0, 0, 0)),
            pl.BlockSpec((c_out, 1), lambda i: (0, 0)),
            pl.BlockSpec((k, c_out, c_out), lambda i: (0, 0, 0)),
            pl.BlockSpec((c_out, c_in), lambda i: (0, 0)),
            pl.BlockSpec((c_out, 1), lambda i: (0, 0)),
            pl.BlockSpec((c_out, 1), lambda i: (0, 0)),
        ],
        out_specs=pl.BlockSpec((1, c_out, l), lambda i: (i, 0, 0)),
        compiler_params=pltpu.CompilerParams(dimension_semantics=("parallel",)),
    )(x, w1, b1c, w2, wd, b2c, bdc)


# speedup vs baseline: 2.4405x; 1.1641x over previous
import functools

import jax
import jax.numpy as jnp
from jax.experimental import pallas as pl
from jax.experimental.pallas import tpu as pltpu

_BB = 4


def _wn(v, g):
    norm = jnp.sqrt(jnp.sum(v.astype(jnp.float32) ** 2, axis=(1, 2), keepdims=True))
    return (g.astype(jnp.float32) * v.astype(jnp.float32) / norm)


def _shift_right(a, s):
    if s == 0:
        return a
    rows = a.shape[0]
    return jnp.concatenate(
        [jnp.zeros((rows, s), a.dtype), a[:, :a.shape[1] - s]], axis=1)


def _causal_tap_matmul(w_ref, a, kernel_size, dilation):
    pad = dilation * (kernel_size - 1)
    acc = jnp.dot(w_ref[kernel_size - 1], a, preferred_element_type=jnp.float32)
    for k in range(kernel_size - 1):
        acc = acc + jnp.dot(w_ref[k], _shift_right(a, pad - k * dilation),
                            preferred_element_type=jnp.float32)
    return acc


def _fused_block_kernel(x_ref, w1d_ref, w2_ref, b_ref, o_ref,
                        *, kernel_size, dilation):
    for b in range(x_ref.shape[0]):
        xb = x_ref[b].astype(jnp.bfloat16)

        acc1 = _causal_tap_matmul(w1d_ref, xb, kernel_size, dilation)
        h1 = jnp.maximum(acc1 + b_ref[0], 0.0).astype(jnp.bfloat16)

        acc2 = _causal_tap_matmul(w2_ref, h1, kernel_size, dilation)
        h2 = jnp.maximum(acc2 + b_ref[1], 0.0)

        res = jnp.dot(w1d_ref[kernel_size], xb, preferred_element_type=jnp.float32)
        o_ref[b] = jnp.maximum(h2 + res + b_ref[2], 0.0).astype(o_ref.dtype)


def kernel(x, v1, g1, b1, v2, g2, b2, w_down, b_down):
    n, c_in, l = x.shape
    c_out = v1.shape[0]
    k = v1.shape[2]
    dilation = 2

    w1 = jnp.transpose(_wn(v1, g1), (2, 0, 1))
    w2 = jnp.transpose(_wn(v2, g2), (2, 0, 1)).astype(jnp.bfloat16)
    w1d = jnp.concatenate(
        [w1, w_down.reshape(1, c_out, c_in)], axis=0).astype(jnp.bfloat16)
    bs = jnp.stack([b1, b2, b_down]).astype(jnp.float32).reshape(3, c_out, 1)

    kern = functools.partial(_fused_block_kernel, kernel_size=k,
                             dilation=dilation)
    return pl.pallas_call(
        kern,
        out_shape=jax.ShapeDtypeStruct((n, c_out, l), x.dtype),
        grid=(n // _BB,),
        in_specs=[
            pl.BlockSpec((_BB, c_in, l), lambda i: (i, 0, 0)),
            pl.BlockSpec((k + 1, c_out, c_in), lambda i: (0, 0, 0)),
            pl.BlockSpec((k, c_out, c_out), lambda i: (0, 0, 0)),
            pl.BlockSpec((3, c_out, 1), lambda i: (0, 0, 0)),
        ],
        out_specs=pl.BlockSpec((_BB, c_out, l), lambda i: (i, 0, 0)),
        compiler_params=pltpu.CompilerParams(dimension_semantics=("parallel",)),
    )(x, w1d, w2, bs)
